# Initial kernel scaffold; baseline (speedup 1.0000x reference)
import functools

import jax
import jax.numpy as jnp
from jax import lax
from jax.experimental import pallas as pl
from jax.experimental.pallas import tpu as pltpu
from jax.experimental.pallas import tpu_sc as plsc

NNODES = 10000
NCLASSES = 64
NITER = 5
ALPHA = 0.9
BATCH = 1024
GAMMA = 0.5
D_FEAT = 128
NDIM = 128
N_EDGES = 320000

NE_PAD = 327680
HALF = NCLASSES // 2
NS = 16
EPT = NE_PAD // NS
CHUNK = 1024
NCHUNK = EPT // CHUNK
GSUB = CHUNK // 128
NPT = NNODES // NS

C1 = ALPHA * NNODES / (2.0 * BATCH)
C2 = 1.0 - ALPHA


def _sc_body(logit_hbm, col_hbm, row_hbm, w_hbm, steps_hbm, z_hbm,
             colv, rowv, wv, rows_v, zv, lv, accv, stepsv, acc_sh, sem):
    c = lax.axis_index("c")
    s = lax.axis_index("s")
    nb = s * NPT
    zb = c * NNODES + nb

    zero16 = jnp.zeros((16,), jnp.float32)

    pltpu.sync_copy(steps_hbm, stepsv)
    pltpu.sync_copy(logit_hbm.at[pl.ds(zb, NPT)], lv)
    pltpu.sync_copy(lv, z_hbm.at[pl.ds(zb, NPT)])

    @pl.loop(0, NPT * 2)
    def _zero_init(v):
        r = v >> 1
        q = (v & 1) * 16
        accv[r, pl.ds(q, 16)] = zero16

    pltpu.sync_copy(accv, acc_sh.at[pl.ds(nb, NPT)])
    plsc.subcore_barrier()

    @pl.loop(0, NITER)
    def _iter(j):
        @pl.loop(0, NCHUNK)
        def _chunk(t):
            rb = s * (EPT // 128) + t * GSUB
            pltpu.sync_copy(col_hbm.at[c, pl.ds(rb, GSUB)], colv)
            pltpu.sync_copy(row_hbm.at[pl.ds(rb, GSUB)], rowv)
            pltpu.sync_copy(w_hbm.at[pl.ds(rb * 128, CHUNK)], wv)
            cps = [
                pltpu.async_copy(
                    z_hbm.at[colv.at[g]],
                    rows_v.at[pl.ds(g * 128, 128)], sem)
                for g in range(GSUB)
            ]
            for cp in cps:
                cp.wait()

            @pl.loop(0, CHUNK)
            def _scale(e):
                es = jnp.full((16,), e, jnp.int32)
                wb = plsc.load_gather(wv, [es])
                r0 = rows_v[e, pl.ds(0, 16)]
                rows_v[e, pl.ds(0, 16)] = r0 * wb
                r1 = rows_v[e, pl.ds(16, 16)]
                rows_v[e, pl.ds(16, 16)] = r1 * wb

            for g in range(GSUB):
                pltpu.sync_copy(rows_v.at[pl.ds(g * 128, 128)],
                                acc_sh.at[rowv.at[g]], add=True)

        plsc.subcore_barrier()

        pltpu.sync_copy(acc_sh.at[pl.ds(nb, NPT)], accv)
        pltpu.sync_copy(z_hbm.at[pl.ds(zb, NPT)], zv)
        stepj = plsc.load_gather(stepsv, [jnp.full((16,), j, jnp.int32)])
        c1v = jnp.full((16,), C1, jnp.float32)
        c2v = jnp.full((16,), C2, jnp.float32)

        @pl.loop(0, NPT * 2)
        def _upd(v):
            r = v >> 1
            q = (v & 1) * 16
            z = zv[r, pl.ds(q, 16)]
            a = accv[r, pl.ds(q, 16)]
            l = lv[r, pl.ds(q, 16)]
            zv[r, pl.ds(q, 16)] = z + stepj * (c1v * a - z + c2v * l)

        pltpu.sync_copy(zv, z_hbm.at[pl.ds(zb, NPT)])

        @pl.loop(0, NPT * 2)
        def _zero(v):
            r = v >> 1
            q = (v & 1) * 16
            accv[r, pl.ds(q, 16)] = zero16

        pltpu.sync_copy(accv, acc_sh.at[pl.ds(nb, NPT)])
        plsc.subcore_barrier()


def _dense_body(x_ref, wm_ref, bm_ref, wo_ref, bo_ref, o_ref):
    h = jnp.dot(x_ref[...], wm_ref[...], preferred_element_type=jnp.float32)
    h = jnp.maximum(h + bm_ref[...], 0.0)
    o_ref[...] = (jnp.dot(h, wo_ref[...], preferred_element_type=jnp.float32)
                  + bo_ref[...])


def _softmax_body(za_ref, zb_ref, o_ref):
    x = jnp.concatenate([za_ref[...], zb_ref[...]], axis=1)
    m = jnp.max(x, axis=1, keepdims=True)
    e = jnp.exp(x - m)
    o_ref[...] = e / jnp.sum(e, axis=1, keepdims=True)


def kernel(Xr, edge_index, edge_weight, W_mean, b_mean, W_out, b_out, i):
    row = edge_index[0].astype(jnp.int32)
    col = edge_index[1].astype(jnp.int32)
    w = edge_weight.astype(jnp.float32)

    pad = NE_PAD - N_EDGES
    colp = jnp.concatenate([col, jnp.zeros((pad,), jnp.int32)])
    rowp = jnp.concatenate([row, jnp.zeros((pad,), jnp.int32)])
    wp = jnp.concatenate([w, jnp.zeros((pad,), jnp.float32)])
    col3 = jnp.stack([colp, colp + NNODES]).reshape(2, NE_PAD // 128, 128)
    row2 = rowp.reshape(NE_PAD // 128, 128)

    ii = jnp.asarray(i, jnp.float32)
    steps = (1.0 + ii + jnp.arange(NITER, dtype=jnp.float32)) ** (-GAMMA)
    steps16 = jnp.concatenate([steps, jnp.zeros((16 - NITER,), jnp.float32)])

    bm2 = b_mean.reshape(1, D_FEAT)
    bo2 = b_out.reshape(1, NCLASSES)

    logit = pl.pallas_call(
        _dense_body,
        grid=(10, 2),
        in_specs=[
            pl.BlockSpec((1000, D_FEAT), lambda m, h: (m, 0)),
            pl.BlockSpec((D_FEAT, NDIM), lambda m, h: (0, 0)),
            pl.BlockSpec((1, NDIM), lambda m, h: (0, 0)),
            pl.BlockSpec((NDIM, HALF), lambda m, h: (0, h)),
            pl.BlockSpec((1, HALF), lambda m, h: (0, h)),
        ],
        out_specs=pl.BlockSpec((1000, HALF), lambda m, h: (h * 10 + m, 0)),
        out_shape=jax.ShapeDtypeStruct((2 * NNODES, HALF), jnp.float32),
    )(Xr, W_mean, bm2, W_out, bo2)

    mesh = plsc.VectorSubcoreMesh(core_axis_name="c", subcore_axis_name="s")
    zf = pl.kernel(
        _sc_body,
        out_type=jax.ShapeDtypeStruct((2 * NNODES, HALF), jnp.float32),
        mesh=mesh,
        scratch_types=[
            pltpu.VMEM((GSUB, 128), jnp.int32),
            pltpu.VMEM((GSUB, 128), jnp.int32),
            pltpu.VMEM((CHUNK,), jnp.float32),
            pltpu.VMEM((CHUNK, HALF), jnp.float32),
            pltpu.VMEM((NPT, HALF), jnp.float32),
            pltpu.VMEM((NPT, HALF), jnp.float32),
            pltpu.VMEM((NPT, HALF), jnp.float32),
            pltpu.VMEM((16,), jnp.float32),
            pltpu.VMEM_SHARED((NNODES, HALF), jnp.float32),
            pltpu.SemaphoreType.DMA,
        ],
    )(logit, col3, row2, wp, steps16)

    out = pl.pallas_call(
        _softmax_body,
        grid=(10,),
        in_specs=[
            pl.BlockSpec((1000, HALF), lambda m: (m, 0)),
            pl.BlockSpec((1000, HALF), lambda m: (m + 10, 0)),
        ],
        out_specs=pl.BlockSpec((1000, NCLASSES), lambda m: (m, 0)),
        out_shape=jax.ShapeDtypeStruct((NNODES, NCLASSES), jnp.float32),
    )(zf, zf)
    return out


# speedup vs baseline: 5.7118x; 5.7118x over previous
import functools

import jax
import jax.numpy as jnp
from jax import lax
from jax.experimental import pallas as pl
from jax.experimental.pallas import tpu as pltpu
from jax.experimental.pallas import tpu_sc as plsc

NNODES = 10000
NCLASSES = 64
NITER = 5
ALPHA = 0.9
BATCH = 1024
GAMMA = 0.5
D_FEAT = 128
NDIM = 128
N_EDGES = 320000

NE_PAD = 327680
HALF = NCLASSES // 2
NS = 16
EPT = NE_PAD // NS
CHUNK = 1024
NCHUNK = EPT // CHUNK
GSUB = CHUNK // 128
NP_PAD = 10240
NPT = NP_PAD // NS

C1 = ALPHA * NNODES / (2.0 * BATCH)
C2 = 1.0 - ALPHA


def _sc_body(logit_hbm, col_hbm, row_hbm, w_hbm, steps_hbm, z_hbm,
             colv, rowv, wv, rows_v, zv, lv, accv, stepsv, acc_sh, sem):
    c = lax.axis_index("c")
    s = lax.axis_index("s")
    nb = s * NPT
    zb = c * NP_PAD + nb

    zero16 = jnp.zeros((16,), jnp.float32)

    pltpu.sync_copy(steps_hbm, stepsv)
    pltpu.sync_copy(logit_hbm.at[pl.ds(zb, NPT)], lv)
    pltpu.sync_copy(lv, z_hbm.at[pl.ds(zb, NPT)])

    @pl.loop(0, NPT * 2)
    def _zero_init(v):
        r = v >> 1
        q = (v & 1) * 16
        accv[r, pl.ds(q, 16)] = zero16

    pltpu.sync_copy(accv, acc_sh.at[pl.ds(nb, NPT)])
    plsc.subcore_barrier()

    steps_vec = stepsv[...]

    for j in range(NITER):
        @pl.loop(0, NCHUNK)
        def _chunk(t):
            rb = s * (EPT // 128) + t * GSUB
            pltpu.sync_copy(col_hbm.at[c, pl.ds(rb, GSUB)], colv)
            pltpu.sync_copy(row_hbm.at[pl.ds(rb, GSUB)], rowv)
            pltpu.sync_copy(w_hbm.at[pl.ds(rb * 128, CHUNK)], wv)
            cps = [
                pltpu.async_copy(
                    z_hbm.at[colv.at[g]],
                    rows_v.at[pl.ds(g * 128, 128)], sem)
                for g in range(GSUB)
            ]
            for cp in cps:
                cp.wait()

            @pl.loop(0, CHUNK // 16)
            def _scale(e16):
                wvec = wv[pl.ds(e16 * 16, 16)]
                for k in range(16):
                    e = e16 * 16 + k
                    wb = jnp.full((16,), wvec[k])
                    r0 = rows_v[e, pl.ds(0, 16)]
                    rows_v[e, pl.ds(0, 16)] = r0 * wb
                    r1 = rows_v[e, pl.ds(16, 16)]
                    rows_v[e, pl.ds(16, 16)] = r1 * wb

            for g in range(GSUB):
                pltpu.sync_copy(rows_v.at[pl.ds(g * 128, 128)],
                                acc_sh.at[rowv.at[g]], add=True)

        plsc.subcore_barrier()

        pltpu.sync_copy(acc_sh.at[pl.ds(nb, NPT)], accv)
        pltpu.sync_copy(z_hbm.at[pl.ds(zb, NPT)], zv)
        stepj = jnp.full((16,), steps_vec[j])
        c1v = jnp.full((16,), C1, jnp.float32)
        c2v = jnp.full((16,), C2, jnp.float32)

        @pl.loop(0, NPT * 2)
        def _upd(v):
            r = v >> 1
            q = (v & 1) * 16
            z = zv[r, pl.ds(q, 16)]
            a = accv[r, pl.ds(q, 16)]
            l = lv[r, pl.ds(q, 16)]
            zv[r, pl.ds(q, 16)] = z + stepj * (c1v * a - z + c2v * l)

        pltpu.sync_copy(zv, z_hbm.at[pl.ds(zb, NPT)])

        @pl.loop(0, NPT * 2)
        def _zero(v):
            r = v >> 1
            q = (v & 1) * 16
            accv[r, pl.ds(q, 16)] = zero16

        pltpu.sync_copy(accv, acc_sh.at[pl.ds(nb, NPT)])
        plsc.subcore_barrier()


def _dense_body(x_ref, wm_ref, bm_ref, wo_ref, bo_ref, o_ref):
    h = jnp.dot(x_ref[...], wm_ref[...], preferred_element_type=jnp.float32)
    h = jnp.maximum(h + bm_ref[...], 0.0)
    o_ref[...] = (jnp.dot(h, wo_ref[0], preferred_element_type=jnp.float32)
                  + bo_ref[0])


def _softmax_body(za_ref, zb_ref, o_ref):
    x = jnp.concatenate([za_ref[...], zb_ref[...]], axis=1)
    m = jnp.max(x, axis=1, keepdims=True)
    e = jnp.exp(x - m)
    o_ref[...] = e / jnp.sum(e, axis=1, keepdims=True)


def kernel(Xr, edge_index, edge_weight, W_mean, b_mean, W_out, b_out, i):
    row = edge_index[0].astype(jnp.int32)
    col = edge_index[1].astype(jnp.int32)
    w = edge_weight.astype(jnp.float32)

    pad = NE_PAD - N_EDGES
    colp = jnp.concatenate([col, jnp.zeros((pad,), jnp.int32)])
    rowp = jnp.concatenate([row, jnp.zeros((pad,), jnp.int32)])
    wp = jnp.concatenate([w, jnp.zeros((pad,), jnp.float32)])
    col3 = jnp.stack([colp, colp + NP_PAD]).reshape(2, NE_PAD // 128, 128)
    row2 = rowp.reshape(NE_PAD // 128, 128)

    ii = jnp.asarray(i, jnp.float32)
    steps = (1.0 + ii + jnp.arange(NITER, dtype=jnp.float32)) ** (-GAMMA)
    steps16 = jnp.concatenate([steps, jnp.zeros((16 - NITER,), jnp.float32)])

    bm2 = b_mean.reshape(1, D_FEAT)
    wo3 = jnp.stack([W_out[:, :HALF], W_out[:, HALF:]])
    bo3 = b_out.reshape(2, 1, HALF)

    xr_pad = jnp.concatenate(
        [Xr, jnp.zeros((NP_PAD - NNODES, D_FEAT), jnp.float32)])
    logit = pl.pallas_call(
        _dense_body,
        grid=(16, 2),
        in_specs=[
            pl.BlockSpec((NPT, D_FEAT), lambda m, h: (m, 0)),
            pl.BlockSpec((D_FEAT, NDIM), lambda m, h: (0, 0)),
            pl.BlockSpec((1, NDIM), lambda m, h: (0, 0)),
            pl.BlockSpec((1, NDIM, HALF), lambda m, h: (h, 0, 0)),
            pl.BlockSpec((1, 1, HALF), lambda m, h: (h, 0, 0)),
        ],
        out_specs=pl.BlockSpec((NPT, HALF), lambda m, h: (h * 16 + m, 0)),
        out_shape=jax.ShapeDtypeStruct((2 * NP_PAD, HALF), jnp.float32),
    )(xr_pad, W_mean, bm2, wo3, bo3)

    mesh = plsc.VectorSubcoreMesh(core_axis_name="c", subcore_axis_name="s")
    zf = pl.kernel(
        _sc_body,
        out_type=jax.ShapeDtypeStruct((2 * NP_PAD, HALF), jnp.float32),
        mesh=mesh,
        compiler_params=pltpu.CompilerParams(use_tc_tiling_on_sc=False),
        scratch_types=[
            pltpu.VMEM((GSUB, 128), jnp.int32),
            pltpu.VMEM((GSUB, 128), jnp.int32),
            pltpu.VMEM((CHUNK,), jnp.float32),
            pltpu.VMEM((CHUNK, HALF), jnp.float32),
            pltpu.VMEM((NPT, HALF), jnp.float32),
            pltpu.VMEM((NPT, HALF), jnp.float32),
            pltpu.VMEM((NPT, HALF), jnp.float32),
            pltpu.VMEM((16,), jnp.float32),
            pltpu.VMEM_SHARED((NP_PAD, HALF), jnp.float32),
            pltpu.SemaphoreType.DMA,
        ],
    )(logit, col3, row2, wp, steps16)

    out = pl.pallas_call(
        _softmax_body,
        grid=(125,),
        in_specs=[
            pl.BlockSpec((80, HALF), lambda m: (m, 0)),
            pl.BlockSpec((80, HALF), lambda m: (m + NP_PAD // 80, 0)),
        ],
        out_specs=pl.BlockSpec((80, NCLASSES), lambda m: (m, 0)),
        out_shape=jax.ShapeDtypeStruct((NNODES, NCLASSES), jnp.float32),
    )(zf, zf)
    return out


# speedup vs baseline: 7.4142x; 1.2980x over previous
import functools

import jax
import jax.numpy as jnp
from jax import lax
from jax.experimental import pallas as pl
from jax.experimental.pallas import tpu as pltpu
from jax.experimental.pallas import tpu_sc as plsc

NNODES = 10000
NCLASSES = 64
NITER = 5
ALPHA = 0.9
BATCH = 1024
GAMMA = 0.5
D_FEAT = 128
NDIM = 128
N_EDGES = 320000

NE_PAD = 327680
HALF = NCLASSES // 2
NS = 16
EPT = NE_PAD // NS
CHUNK = 1024
NCHUNK = EPT // CHUNK
GSUB = CHUNK // 128
NP_PAD = 10240
NPT = NP_PAD // NS
SUBB = 160

C1 = ALPHA * NNODES / (2.0 * BATCH)
C2 = 1.0 - ALPHA


def _sc_body(logit_hbm, col_hbm, row_hbm, w_hbm, steps_hbm, z_hbm,
             colv0, rowv0, wv0, rows0, colv1, rowv1, wv1, rows1,
             zv, lv, accv, stepsv,
             semi0, semg0, sems0, semi1, semg1, sems1, acc_sh):
    c = lax.axis_index("c")
    s = lax.axis_index("s")
    nb = s * NPT
    zb = c * NP_PAD + nb
    RPT = EPT // 128

    idx = [(colv0, rowv0, wv0, semi0), (colv1, rowv1, wv1, semi1)]
    rows = [(rows0, semg0, sems0), (rows1, semg1, sems1)]

    zero16 = jnp.zeros((16,), jnp.float32)

    def fire_idx(t, p):
        colb, rowb, wb, sem = idx[p]
        rb = s * RPT + t * GSUB
        pltpu.async_copy(col_hbm.at[c, pl.ds(rb, GSUB)], colb, sem)
        pltpu.async_copy(row_hbm.at[pl.ds(rb, GSUB)], rowb, sem)
        pltpu.async_copy(w_hbm.at[pl.ds(rb * 128, CHUNK)], wb, sem)

    def wait_idx(p):
        colb, rowb, wb, sem = idx[p]
        pltpu.make_async_copy(col_hbm.at[c, pl.ds(0, GSUB)], colb, sem).wait()
        pltpu.make_async_copy(row_hbm.at[pl.ds(0, GSUB)], rowb, sem).wait()
        pltpu.make_async_copy(w_hbm.at[pl.ds(0, CHUNK)], wb, sem).wait()

    def fire_gather(p):
        colb = idx[p][0]
        rowsb, semg, _ = rows[p]
        for g in range(GSUB):
            pltpu.async_copy(z_hbm.at[colb.at[g]],
                             rowsb.at[pl.ds(g * 128, 128)], semg)

    def wait_gather(p):
        rowsb, semg, _ = rows[p]
        pltpu.make_async_copy(z_hbm.at[pl.ds(0, CHUNK)], rowsb, semg).wait()

    def fire_scatter(p):
        rowb = idx[p][1]
        rowsb, _, sems = rows[p]
        for g in range(GSUB):
            pltpu.async_copy(rowsb.at[pl.ds(g * 128, 128)],
                             acc_sh.at[rowb.at[g]], sems, add=True)

    def wait_scatter(p):
        rowsb, _, sems = rows[p]
        pltpu.make_async_copy(rowsb, acc_sh.at[pl.ds(0, CHUNK)], sems).wait()

    def scale(p):
        wb_ref = idx[p][2]
        rowsb = rows[p][0]

        @pl.loop(0, CHUNK // 16)
        def _scale(e16):
            wvec = wb_ref[pl.ds(e16 * 16, 16)]
            for k in range(16):
                e = e16 * 16 + k
                wbk = jnp.full((16,), wvec[k])
                r0 = rowsb[e, pl.ds(0, 16)]
                rowsb[e, pl.ds(0, 16)] = r0 * wbk
                r1 = rowsb[e, pl.ds(16, 16)]
                rowsb[e, pl.ds(16, 16)] = r1 * wbk

    pltpu.sync_copy(logit_hbm.at[pl.ds(zb, NPT)], lv)
    pltpu.sync_copy(lv, z_hbm.at[pl.ds(zb, NPT)])

    @pl.loop(0, SUBB * 2)
    def _zero_init(v):
        r = v >> 1
        q = (v & 1) * 16
        accv[r, pl.ds(q, 16)] = zero16

    pltpu.sync_copy(accv, acc_sh.at[pl.ds(nb, SUBB)])
    pltpu.sync_copy(accv, acc_sh.at[pl.ds(nb + SUBB, SUBB)])
    plsc.subcore_barrier()

    c1v = jnp.full((16,), C1, jnp.float32)
    c2v = jnp.full((16,), C2, jnp.float32)

    @pl.loop(0, NITER)
    def _iter(j):
        fire_idx(0, 0)
        wait_idx(0)
        fire_gather(0)
        for t in range(NCHUNK):
            p = t % 2
            q = 1 - p
            if t >= 1:
                wait_scatter(q)
            if t + 1 < NCHUNK:
                fire_idx(t + 1, q)
                wait_idx(q)
                fire_gather(q)
            wait_gather(p)
            scale(p)
            fire_scatter(p)
        wait_scatter((NCHUNK - 1) % 2)
        plsc.subcore_barrier()

        pltpu.sync_copy(steps_hbm.at[pl.ds(j * 16, 16)], stepsv)
        stepj = stepsv[...]
        for u in range(NPT // SUBB):
            off = nb + u * SUBB
            zoff = zb + u * SUBB
            pltpu.sync_copy(acc_sh.at[pl.ds(off, SUBB)], accv)
            pltpu.sync_copy(z_hbm.at[pl.ds(zoff, SUBB)], zv)

            @pl.loop(0, SUBB * 2)
            def _upd(v):
                r = v >> 1
                qq = (v & 1) * 16
                z = zv[r, pl.ds(qq, 16)]
                a = accv[r, pl.ds(qq, 16)]
                l = lv[r + u * SUBB, pl.ds(qq, 16)]
                zv[r, pl.ds(qq, 16)] = z + stepj * (c1v * a - z + c2v * l)

            pltpu.sync_copy(zv, z_hbm.at[pl.ds(zoff, SUBB)])

            @pl.loop(0, SUBB * 2)
            def _zero(v):
                r = v >> 1
                qq = (v & 1) * 16
                accv[r, pl.ds(qq, 16)] = zero16

            pltpu.sync_copy(accv, acc_sh.at[pl.ds(off, SUBB)])
        plsc.subcore_barrier()


def _dense_body(x_ref, wm_ref, bm_ref, wo_ref, bo_ref, o_ref):
    h = jnp.dot(x_ref[...], wm_ref[...], preferred_element_type=jnp.float32)
    h = jnp.maximum(h + bm_ref[...], 0.0)
    o_ref[...] = (jnp.dot(h, wo_ref[0], preferred_element_type=jnp.float32)
                  + bo_ref[0])


def _softmax_body(za_ref, zb_ref, o_ref):
    x = jnp.concatenate([za_ref[...], zb_ref[...]], axis=1)
    m = jnp.max(x, axis=1, keepdims=True)
    e = jnp.exp(x - m)
    o_ref[...] = e / jnp.sum(e, axis=1, keepdims=True)


def kernel(Xr, edge_index, edge_weight, W_mean, b_mean, W_out, b_out, i):
    row = edge_index[0].astype(jnp.int32)
    col = edge_index[1].astype(jnp.int32)
    w = edge_weight.astype(jnp.float32)

    pad = NE_PAD - N_EDGES
    colp = jnp.concatenate([col, jnp.zeros((pad,), jnp.int32)])
    rowp = jnp.concatenate([row, jnp.zeros((pad,), jnp.int32)])
    wp = jnp.concatenate([w, jnp.zeros((pad,), jnp.float32)])
    col3 = jnp.stack([colp, colp + NP_PAD]).reshape(2, NE_PAD // 128, 128)
    row2 = rowp.reshape(NE_PAD // 128, 128)

    ii = jnp.asarray(i, jnp.float32)
    steps = (1.0 + ii + jnp.arange(NITER, dtype=jnp.float32)) ** (-GAMMA)
    steps16 = jnp.repeat(steps, 16)

    bm2 = b_mean.reshape(1, D_FEAT)
    wo3 = jnp.stack([W_out[:, :HALF], W_out[:, HALF:]])
    bo3 = b_out.reshape(2, 1, HALF)

    xr_pad = jnp.concatenate(
        [Xr, jnp.zeros((NP_PAD - NNODES, D_FEAT), jnp.float32)])
    logit = pl.pallas_call(
        _dense_body,
        grid=(16, 2),
        in_specs=[
            pl.BlockSpec((NPT, D_FEAT), lambda m, h: (m, 0)),
            pl.BlockSpec((D_FEAT, NDIM), lambda m, h: (0, 0)),
            pl.BlockSpec((1, NDIM), lambda m, h: (0, 0)),
            pl.BlockSpec((1, NDIM, HALF), lambda m, h: (h, 0, 0)),
            pl.BlockSpec((1, 1, HALF), lambda m, h: (h, 0, 0)),
        ],
        out_specs=pl.BlockSpec((NPT, HALF), lambda m, h: (h * 16 + m, 0)),
        out_shape=jax.ShapeDtypeStruct((2 * NP_PAD, HALF), jnp.float32),
    )(xr_pad, W_mean, bm2, wo3, bo3)

    mesh = plsc.VectorSubcoreMesh(core_axis_name="c", subcore_axis_name="s")
    zf = pl.kernel(
        _sc_body,
        out_type=jax.ShapeDtypeStruct((2 * NP_PAD, HALF), jnp.float32),
        mesh=mesh,
        compiler_params=pltpu.CompilerParams(use_tc_tiling_on_sc=False),
        scratch_types=[
            pltpu.VMEM((GSUB, 128), jnp.int32),
            pltpu.VMEM((GSUB, 128), jnp.int32),
            pltpu.VMEM((CHUNK,), jnp.float32),
            pltpu.VMEM((CHUNK, HALF), jnp.float32),
            pltpu.VMEM((GSUB, 128), jnp.int32),
            pltpu.VMEM((GSUB, 128), jnp.int32),
            pltpu.VMEM((CHUNK,), jnp.float32),
            pltpu.VMEM((CHUNK, HALF), jnp.float32),
            pltpu.VMEM((SUBB, HALF), jnp.float32),
            pltpu.VMEM((NPT, HALF), jnp.float32),
            pltpu.VMEM((SUBB, HALF), jnp.float32),
            pltpu.VMEM((16,), jnp.float32),
            pltpu.SemaphoreType.DMA,
            pltpu.SemaphoreType.DMA,
            pltpu.SemaphoreType.DMA,
            pltpu.SemaphoreType.DMA,
            pltpu.SemaphoreType.DMA,
            pltpu.SemaphoreType.DMA,
            pltpu.VMEM_SHARED((NP_PAD, HALF), jnp.float32),
        ],
    )(logit, col3, row2, wp, steps16)

    out = pl.pallas_call(
        _softmax_body,
        grid=(125,),
        in_specs=[
            pl.BlockSpec((80, HALF), lambda m: (m, 0)),
            pl.BlockSpec((80, HALF), lambda m: (m + NP_PAD // 80, 0)),
        ],
        out_specs=pl.BlockSpec((80, NCLASSES), lambda m: (m, 0)),
        out_shape=jax.ShapeDtypeStruct((NNODES, NCLASSES), jnp.float32),
    )(zf, zf)
    return out


# speedup vs baseline: 11.5432x; 1.5569x over previous
import functools

import jax
import jax.numpy as jnp
from jax import lax
from jax.experimental import pallas as pl
from jax.experimental.pallas import tpu as pltpu
from jax.experimental.pallas import tpu_sc as plsc

NNODES = 10000
NCLASSES = 64
NITER = 5
ALPHA = 0.9
BATCH = 1024
GAMMA = 0.5
D_FEAT = 128
NDIM = 128
N_EDGES = 320000

NE_PAD = 327680
HALF = NCLASSES // 2
NS = 16
EPT = NE_PAD // NS
CHUNK = 1024
NCHUNK = EPT // CHUNK
GSUB = CHUNK // 128
NP_PAD = 10240
NPT = NP_PAD // NS
SUBB = 160

C1 = ALPHA * NNODES / (2.0 * BATCH)
C2 = 1.0 - ALPHA


def _sc_body(logit_hbm, col_hbm, row_hbm, w_hbm, steps_hbm, z_hbm,
             colv0, rowv0, wv0, rows0, colv1, rowv1, wv1, rows1,
             zv, lv, accv, stepsv,
             semi0, semg0, sems0, semi1, semg1, sems1, acc_sh):
    c = lax.axis_index("c")
    s = lax.axis_index("s")
    nb = s * NPT
    zb = c * NP_PAD + nb
    RPT = EPT // 128

    idx = [(colv0, rowv0, wv0, semi0), (colv1, rowv1, wv1, semi1)]
    rows = [(rows0, semg0, sems0), (rows1, semg1, sems1)]

    zero16 = jnp.zeros((16,), jnp.float32)

    def fire_idx(t, p):
        colb, rowb, wb, sem = idx[p]
        rb = s * RPT + t * GSUB
        pltpu.async_copy(col_hbm.at[c, pl.ds(rb, GSUB)], colb, sem)
        pltpu.async_copy(row_hbm.at[pl.ds(rb, GSUB)], rowb, sem)
        pltpu.async_copy(w_hbm.at[pl.ds(rb * 128, CHUNK)], wb, sem)

    def wait_idx(p):
        colb, rowb, wb, sem = idx[p]
        pltpu.make_async_copy(col_hbm.at[c, pl.ds(0, GSUB)], colb, sem).wait()
        pltpu.make_async_copy(row_hbm.at[pl.ds(0, GSUB)], rowb, sem).wait()
        pltpu.make_async_copy(w_hbm.at[pl.ds(0, CHUNK)], wb, sem).wait()

    def fire_gather(p):
        colb = idx[p][0]
        rowsb, semg, _ = rows[p]
        for g in range(GSUB):
            pltpu.async_copy(z_hbm.at[colb.at[g]],
                             rowsb.at[pl.ds(g * 128, 128)], semg)

    def wait_gather(p):
        rowsb, semg, _ = rows[p]
        pltpu.make_async_copy(z_hbm.at[pl.ds(0, CHUNK)], rowsb, semg).wait()

    def fire_scatter(p):
        rowb = idx[p][1]
        rowsb, _, sems = rows[p]
        for g in range(GSUB):
            pltpu.async_copy(rowsb.at[pl.ds(g * 128, 128)],
                             acc_sh.at[rowb.at[g]], sems, add=True)

    def wait_scatter(p):
        rowsb, _, sems = rows[p]
        pltpu.make_async_copy(rowsb, acc_sh.at[pl.ds(0, CHUNK)], sems).wait()

    def scale(p):
        wb_ref = idx[p][2]
        rowsb = rows[p][0]

        @pl.loop(0, CHUNK // 16)
        def _scale(e16):
            wvec = wb_ref[pl.ds(e16 * 16, 16)]
            for k in range(16):
                e = e16 * 16 + k
                wbk = jnp.full((16,), wvec[k])
                r0 = rowsb[e, pl.ds(0, 16)]
                rowsb[e, pl.ds(0, 16)] = r0 * wbk
                r1 = rowsb[e, pl.ds(16, 16)]
                rowsb[e, pl.ds(16, 16)] = r1 * wbk

    pltpu.sync_copy(logit_hbm.at[pl.ds(zb, NPT)], lv)
    pltpu.sync_copy(lv, z_hbm.at[pl.ds(zb, NPT)])

    @pl.loop(0, SUBB * 2)
    def _zero_init(v):
        r = v >> 1
        q = (v & 1) * 16
        accv[r, pl.ds(q, 16)] = zero16

    pltpu.sync_copy(accv, acc_sh.at[pl.ds(nb, SUBB)])
    pltpu.sync_copy(accv, acc_sh.at[pl.ds(nb + SUBB, SUBB)])
    plsc.subcore_barrier()

    c1v = jnp.full((16,), C1, jnp.float32)
    c2v = jnp.full((16,), C2, jnp.float32)

    @pl.loop(0, NITER)
    def _iter(j):
        fire_idx(0, 0)
        wait_idx(0)
        fire_gather(0)
        for t in range(NCHUNK):
            p = t % 2
            q = 1 - p
            if t >= 1:
                wait_scatter(q)
            if t + 1 < NCHUNK:
                fire_idx(t + 1, q)
                wait_idx(q)
                fire_gather(q)
            wait_gather(p)
            scale(p)
            fire_scatter(p)
        wait_scatter((NCHUNK - 1) % 2)
        plsc.subcore_barrier()

        pltpu.sync_copy(steps_hbm.at[pl.ds(j * 16, 16)], stepsv)
        stepj = stepsv[...]
        for u in range(NPT // SUBB):
            off = nb + u * SUBB
            zoff = zb + u * SUBB
            pltpu.sync_copy(acc_sh.at[pl.ds(off, SUBB)], accv)
            pltpu.sync_copy(z_hbm.at[pl.ds(zoff, SUBB)], zv)

            @pl.loop(0, SUBB * 2)
            def _upd(v):
                r = v >> 1
                qq = (v & 1) * 16
                z = zv[r, pl.ds(qq, 16)]
                a = accv[r, pl.ds(qq, 16)]
                l = lv[r + u * SUBB, pl.ds(qq, 16)]
                zv[r, pl.ds(qq, 16)] = z + stepj * (c1v * a - z + c2v * l)

            pltpu.sync_copy(zv, z_hbm.at[pl.ds(zoff, SUBB)])

            @pl.loop(0, SUBB * 2)
            def _zero(v):
                r = v >> 1
                qq = (v & 1) * 16
                accv[r, pl.ds(qq, 16)] = zero16

            pltpu.sync_copy(accv, acc_sh.at[pl.ds(off, SUBB)])
        plsc.subcore_barrier()


def _dense_body(x_ref, wm_ref, bm_ref, wo_ref, bo_ref, o_ref):
    h = jnp.dot(x_ref[...], wm_ref[...], preferred_element_type=jnp.float32)
    h = jnp.maximum(h + bm_ref[...], 0.0)
    o_ref[...] = (jnp.dot(h, wo_ref[0], preferred_element_type=jnp.float32)
                  + bo_ref[0])


def _softmax_body(za_ref, zb_ref, o_ref):
    x = jnp.concatenate([za_ref[...], zb_ref[...]], axis=1)
    m = jnp.max(x, axis=1, keepdims=True)
    e = jnp.exp(x - m)
    o_ref[...] = e / jnp.sum(e, axis=1, keepdims=True)


def kernel(Xr, edge_index, edge_weight, W_mean, b_mean, W_out, b_out, i):
    row = edge_index[0].astype(jnp.int32)
    col = edge_index[1].astype(jnp.int32)
    w = edge_weight.astype(jnp.float32)

    pad = NE_PAD - N_EDGES
    spread = jnp.arange(pad, dtype=jnp.int32) % NNODES
    colp = jnp.concatenate([col, spread])
    rowp = jnp.concatenate([row, spread])
    wp = jnp.concatenate([w, jnp.zeros((pad,), jnp.float32)])
    col3 = jnp.stack([colp, colp + NP_PAD]).reshape(2, NE_PAD // 128, 128)
    row2 = rowp.reshape(NE_PAD // 128, 128)

    ii = jnp.asarray(i, jnp.float32)
    steps = (1.0 + ii + jnp.arange(NITER, dtype=jnp.float32)) ** (-GAMMA)
    steps16 = jnp.repeat(steps, 16)

    bm2 = b_mean.reshape(1, D_FEAT)
    wo3 = jnp.stack([W_out[:, :HALF], W_out[:, HALF:]])
    bo3 = b_out.reshape(2, 1, HALF)

    xr_pad = jnp.concatenate(
        [Xr, jnp.zeros((NP_PAD - NNODES, D_FEAT), jnp.float32)])
    logit = pl.pallas_call(
        _dense_body,
        grid=(16, 2),
        in_specs=[
            pl.BlockSpec((NPT, D_FEAT), lambda m, h: (m, 0)),
            pl.BlockSpec((D_FEAT, NDIM), lambda m, h: (0, 0)),
            pl.BlockSpec((1, NDIM), lambda m, h: (0, 0)),
            pl.BlockSpec((1, NDIM, HALF), lambda m, h: (h, 0, 0)),
            pl.BlockSpec((1, 1, HALF), lambda m, h: (h, 0, 0)),
        ],
        out_specs=pl.BlockSpec((NPT, HALF), lambda m, h: (h * 16 + m, 0)),
        out_shape=jax.ShapeDtypeStruct((2 * NP_PAD, HALF), jnp.float32),
    )(xr_pad, W_mean, bm2, wo3, bo3)

    mesh = plsc.VectorSubcoreMesh(core_axis_name="c", subcore_axis_name="s")
    zf = pl.kernel(
        _sc_body,
        out_type=jax.ShapeDtypeStruct((2 * NP_PAD, HALF), jnp.float32),
        mesh=mesh,
        compiler_params=pltpu.CompilerParams(use_tc_tiling_on_sc=False),
        scratch_types=[
            pltpu.VMEM((GSUB, 128), jnp.int32),
            pltpu.VMEM((GSUB, 128), jnp.int32),
            pltpu.VMEM((CHUNK,), jnp.float32),
            pltpu.VMEM((CHUNK, HALF), jnp.float32),
            pltpu.VMEM((GSUB, 128), jnp.int32),
            pltpu.VMEM((GSUB, 128), jnp.int32),
            pltpu.VMEM((CHUNK,), jnp.float32),
            pltpu.VMEM((CHUNK, HALF), jnp.float32),
            pltpu.VMEM((SUBB, HALF), jnp.float32),
            pltpu.VMEM((NPT, HALF), jnp.float32),
            pltpu.VMEM((SUBB, HALF), jnp.float32),
            pltpu.VMEM((16,), jnp.float32),
            pltpu.SemaphoreType.DMA,
            pltpu.SemaphoreType.DMA,
            pltpu.SemaphoreType.DMA,
            pltpu.SemaphoreType.DMA,
            pltpu.SemaphoreType.DMA,
            pltpu.SemaphoreType.DMA,
            pltpu.VMEM_SHARED((NP_PAD, HALF), jnp.float32),
        ],
    )(logit, col3, row2, wp, steps16)

    out = pl.pallas_call(
        _softmax_body,
        grid=(125,),
        in_specs=[
            pl.BlockSpec((80, HALF), lambda m: (m, 0)),
            pl.BlockSpec((80, HALF), lambda m: (m + NP_PAD // 80, 0)),
        ],
        out_specs=pl.BlockSpec((80, NCLASSES), lambda m: (m, 0)),
        out_shape=jax.ShapeDtypeStruct((NNODES, NCLASSES), jnp.float32),
    )(zf, zf)
    return out


# speedup vs baseline: 12.4825x; 1.0814x over previous
import functools

import jax
import jax.numpy as jnp
from jax import lax
from jax.experimental import pallas as pl
from jax.experimental.pallas import tpu as pltpu
from jax.experimental.pallas import tpu_sc as plsc

NNODES = 10000
NCLASSES = 64
NITER = 5
ALPHA = 0.9
BATCH = 1024
GAMMA = 0.5
D_FEAT = 128
NDIM = 128
N_EDGES = 320000

NE_PAD = 327680
HALF = NCLASSES // 2
NS = 16
EPT = NE_PAD // NS
CHUNK = 1024
NCHUNK = EPT // CHUNK
GSUB = CHUNK // 128
NP_PAD = 10240
NPT = NP_PAD // NS
SUBB = 160

C1 = ALPHA * NNODES / (2.0 * BATCH)
C2 = 1.0 - ALPHA


def _sc_body(logit_hbm, col_hbm, row_hbm, w_hbm, steps_hbm, z_hbm,
             colv0, rowv0, wv0, rows0, colv1, rowv1, wv1, rows1,
             colv2, rowv2, wv2, semi2,
             zv, lv, accv, stepsv,
             semi0, semg0, sems0, semi1, semg1, sems1, acc_sh):
    c = lax.axis_index("c")
    s = lax.axis_index("s")
    nb = s * NPT
    zb = c * NP_PAD + nb
    RPT = EPT // 128

    idx = [(colv0, rowv0, wv0, semi0), (colv1, rowv1, wv1, semi1),
           (colv2, rowv2, wv2, semi2)]
    rows = [(rows0, semg0, sems0), (rows1, semg1, sems1)]

    zero16 = jnp.zeros((16,), jnp.float32)

    def fire_idx(t, p):
        colb, rowb, wb, sem = idx[p]
        rb = s * RPT + t * GSUB
        pltpu.async_copy(col_hbm.at[c, pl.ds(rb, GSUB)], colb, sem)
        pltpu.async_copy(row_hbm.at[pl.ds(rb, GSUB)], rowb, sem)
        pltpu.async_copy(w_hbm.at[pl.ds(rb * 128, CHUNK)], wb, sem)

    def wait_idx(p):
        colb, rowb, wb, sem = idx[p]
        pltpu.make_async_copy(col_hbm.at[c, pl.ds(0, GSUB)], colb, sem).wait()
        pltpu.make_async_copy(row_hbm.at[pl.ds(0, GSUB)], rowb, sem).wait()
        pltpu.make_async_copy(w_hbm.at[pl.ds(0, CHUNK)], wb, sem).wait()

    def fire_gather(r, p):
        colb = idx[r][0]
        rowsb, semg, _ = rows[p]
        for g in range(GSUB):
            pltpu.async_copy(z_hbm.at[colb.at[g]],
                             rowsb.at[pl.ds(g * 128, 128)], semg)

    def wait_gather(p):
        rowsb, semg, _ = rows[p]
        pltpu.make_async_copy(z_hbm.at[pl.ds(0, CHUNK)], rowsb, semg).wait()

    def fire_scatter(r, p):
        rowb = idx[r][1]
        rowsb, _, sems = rows[p]
        for g in range(GSUB):
            pltpu.async_copy(rowsb.at[pl.ds(g * 128, 128)],
                             acc_sh.at[rowb.at[g]], sems, add=True)

    def wait_scatter(p):
        rowsb, _, sems = rows[p]
        pltpu.make_async_copy(rowsb, acc_sh.at[pl.ds(0, CHUNK)], sems).wait()

    def scale(r, p):
        wb_ref = idx[r][2]
        rowsb = rows[p][0]

        @pl.loop(0, CHUNK // 16)
        def _scale(e16):
            wvec = wb_ref[pl.ds(e16 * 16, 16)]
            for k in range(16):
                e = e16 * 16 + k
                wbk = jnp.full((16,), wvec[k])
                r0 = rowsb[e, pl.ds(0, 16)]
                rowsb[e, pl.ds(0, 16)] = r0 * wbk
                r1 = rowsb[e, pl.ds(16, 16)]
                rowsb[e, pl.ds(16, 16)] = r1 * wbk

    pltpu.sync_copy(logit_hbm.at[pl.ds(zb, NPT)], lv)
    pltpu.sync_copy(lv, z_hbm.at[pl.ds(zb, NPT)])

    @pl.loop(0, SUBB * 2)
    def _zero_init(v):
        r = v >> 1
        q = (v & 1) * 16
        accv[r, pl.ds(q, 16)] = zero16

    pltpu.sync_copy(accv, acc_sh.at[pl.ds(nb, SUBB)])
    pltpu.sync_copy(accv, acc_sh.at[pl.ds(nb + SUBB, SUBB)])
    plsc.subcore_barrier()

    c1v = jnp.full((16,), C1, jnp.float32)
    c2v = jnp.full((16,), C2, jnp.float32)

    @pl.loop(0, NITER)
    def _iter(j):
        fire_idx(0, 0)
        fire_idx(1, 1)
        wait_idx(0)
        fire_gather(0, 0)
        for t in range(NCHUNK):
            p = t % 2
            q = 1 - p
            if t >= 1:
                wait_scatter(q)
            if t + 2 < NCHUNK:
                fire_idx(t + 2, (t + 2) % 3)
            if t + 1 < NCHUNK:
                wait_idx((t + 1) % 3)
                fire_gather((t + 1) % 3, q)
            wait_gather(p)
            scale(t % 3, p)
            fire_scatter(t % 3, p)
        wait_scatter((NCHUNK - 1) % 2)
        plsc.subcore_barrier()

        pltpu.sync_copy(steps_hbm.at[pl.ds(j * 16, 16)], stepsv)
        stepj = stepsv[...]
        for u in range(NPT // SUBB):
            off = nb + u * SUBB
            zoff = zb + u * SUBB
            pltpu.sync_copy(acc_sh.at[pl.ds(off, SUBB)], accv)
            pltpu.sync_copy(z_hbm.at[pl.ds(zoff, SUBB)], zv)

            @pl.loop(0, SUBB * 2)
            def _upd(v):
                r = v >> 1
                qq = (v & 1) * 16
                z = zv[r, pl.ds(qq, 16)]
                a = accv[r, pl.ds(qq, 16)]
                l = lv[r + u * SUBB, pl.ds(qq, 16)]
                zv[r, pl.ds(qq, 16)] = z + stepj * (c1v * a - z + c2v * l)

            pltpu.sync_copy(zv, z_hbm.at[pl.ds(zoff, SUBB)])

            @pl.loop(0, SUBB * 2)
            def _zero(v):
                r = v >> 1
                qq = (v & 1) * 16
                accv[r, pl.ds(qq, 16)] = zero16

            pltpu.sync_copy(accv, acc_sh.at[pl.ds(off, SUBB)])
        plsc.subcore_barrier()


def _dense_body(x_ref, wm_ref, bm_ref, wo_ref, bo_ref, o_ref):
    h = jnp.dot(x_ref[...], wm_ref[...], preferred_element_type=jnp.float32)
    h = jnp.maximum(h + bm_ref[...], 0.0)
    o_ref[...] = (jnp.dot(h, wo_ref[0], preferred_element_type=jnp.float32)
                  + bo_ref[0])


def _softmax_body(za_ref, zb_ref, o_ref):
    x = jnp.concatenate([za_ref[...], zb_ref[...]], axis=1)
    m = jnp.max(x, axis=1, keepdims=True)
    e = jnp.exp(x - m)
    o_ref[...] = e / jnp.sum(e, axis=1, keepdims=True)


def kernel(Xr, edge_index, edge_weight, W_mean, b_mean, W_out, b_out, i):
    row = edge_index[0].astype(jnp.int32)
    col = edge_index[1].astype(jnp.int32)
    w = edge_weight.astype(jnp.float32)

    pad = NE_PAD - N_EDGES
    spread = jnp.arange(pad, dtype=jnp.int32) % NNODES
    colp = jnp.concatenate([col, spread])
    rowp = jnp.concatenate([row, spread])
    wp = jnp.concatenate([w, jnp.zeros((pad,), jnp.float32)])
    col3 = jnp.stack([colp, colp + NP_PAD]).reshape(2, NE_PAD // 128, 128)
    row2 = rowp.reshape(NE_PAD // 128, 128)

    ii = jnp.asarray(i, jnp.float32)
    steps = (1.0 + ii + jnp.arange(NITER, dtype=jnp.float32)) ** (-GAMMA)
    steps16 = jnp.repeat(steps, 16)

    bm2 = b_mean.reshape(1, D_FEAT)
    wo3 = jnp.stack([W_out[:, :HALF], W_out[:, HALF:]])
    bo3 = b_out.reshape(2, 1, HALF)

    xr_pad = jnp.concatenate(
        [Xr, jnp.zeros((NP_PAD - NNODES, D_FEAT), jnp.float32)])
    logit = pl.pallas_call(
        _dense_body,
        grid=(16, 2),
        in_specs=[
            pl.BlockSpec((NPT, D_FEAT), lambda m, h: (m, 0)),
            pl.BlockSpec((D_FEAT, NDIM), lambda m, h: (0, 0)),
            pl.BlockSpec((1, NDIM), lambda m, h: (0, 0)),
            pl.BlockSpec((1, NDIM, HALF), lambda m, h: (h, 0, 0)),
            pl.BlockSpec((1, 1, HALF), lambda m, h: (h, 0, 0)),
        ],
        out_specs=pl.BlockSpec((NPT, HALF), lambda m, h: (h * 16 + m, 0)),
        out_shape=jax.ShapeDtypeStruct((2 * NP_PAD, HALF), jnp.float32),
    )(xr_pad, W_mean, bm2, wo3, bo3)

    mesh = plsc.VectorSubcoreMesh(core_axis_name="c", subcore_axis_name="s")
    zf = pl.kernel(
        _sc_body,
        out_type=jax.ShapeDtypeStruct((2 * NP_PAD, HALF), jnp.float32),
        mesh=mesh,
        compiler_params=pltpu.CompilerParams(use_tc_tiling_on_sc=False),
        scratch_types=[
            pltpu.VMEM((GSUB, 128), jnp.int32),
            pltpu.VMEM((GSUB, 128), jnp.int32),
            pltpu.VMEM((CHUNK,), jnp.float32),
            pltpu.VMEM((CHUNK, HALF), jnp.float32),
            pltpu.VMEM((GSUB, 128), jnp.int32),
            pltpu.VMEM((GSUB, 128), jnp.int32),
            pltpu.VMEM((CHUNK,), jnp.float32),
            pltpu.VMEM((CHUNK, HALF), jnp.float32),
            pltpu.VMEM((GSUB, 128), jnp.int32),
            pltpu.VMEM((GSUB, 128), jnp.int32),
            pltpu.VMEM((CHUNK,), jnp.float32),
            pltpu.SemaphoreType.DMA,
            pltpu.VMEM((SUBB, HALF), jnp.float32),
            pltpu.VMEM((NPT, HALF), jnp.float32),
            pltpu.VMEM((SUBB, HALF), jnp.float32),
            pltpu.VMEM((16,), jnp.float32),
            pltpu.SemaphoreType.DMA,
            pltpu.SemaphoreType.DMA,
            pltpu.SemaphoreType.DMA,
            pltpu.SemaphoreType.DMA,
            pltpu.SemaphoreType.DMA,
            pltpu.SemaphoreType.DMA,
            pltpu.VMEM_SHARED((NP_PAD, HALF), jnp.float32),
        ],
    )(logit, col3, row2, wp, steps16)

    out = pl.pallas_call(
        _softmax_body,
        grid=(125,),
        in_specs=[
            pl.BlockSpec((80, HALF), lambda m: (m, 0)),
            pl.BlockSpec((80, HALF), lambda m: (m + NP_PAD // 80, 0)),
        ],
        out_specs=pl.BlockSpec((80, NCLASSES), lambda m: (m, 0)),
        out_shape=jax.ShapeDtypeStruct((NNODES, NCLASSES), jnp.float32),
    )(zf, zf)
    return out
